# baseline (speedup 1.0000x reference)
import functools

import numpy as np
import jax
import jax.numpy as jnp
from jax import lax
from jax.experimental import pallas as pl
from jax.experimental.pallas import tpu as pltpu
from jax.experimental.pallas import tpu_sc as plsc

N_ATTR = 100
N_VAL = 100
ROW = N_ATTR * N_VAL
HALF = ROW // 2
NSEG = N_ATTR // 2
HOUT = NSEG * 2
HCPY = 104
BATCH = 4096
OUT = 2 * N_ATTR
OUTP = 224
L = 16
NTAB = (N_VAL + L - 1) // L
NWORKER = 32
SC_ROWS = 2304
TC_ROWS = BATCH - SC_ROWS
ROWS_PER = SC_ROWS // NWORKER
B_BLK = 128
HPAD = 5008
OPAD = 112
BIGF = np.float32(2.0 ** 30)


def _perm(v, idx):
    return v.at[idx].get(mode="promise_in_bounds")


def _half_compute(buf, tab0, tab1, outbuf):
    lane = lax.iota(jnp.int32, L)
    lanef = lane.astype(jnp.float32)
    half = lane >> 1
    even = (lane & 1) == 0
    si = jnp.zeros((L,), jnp.float32)
    chunk_cache = {}

    def chunk(k):
        if k not in chunk_cache:
            chunk_cache[k] = (buf[pl.ds(L * k, L)], lanef + float(L * k))
        return chunk_cache[k]

    for s in range(NSEG):
        lo = N_VAL * s
        hi = lo + N_VAL
        k0 = lo // L
        k1 = (hi - 1) // L
        m = None
        for k in range(k0, k1 + 1):
            start = L * k
            v, avk = chunk(k)
            full = start >= lo and start + L <= hi
            maskc = None
            if not full:
                maskc = avk >= lo if start < lo else avk < hi
            if m is None:
                if full:
                    m, mi = v, avk
                else:
                    m = jnp.where(maskc, v, -jnp.inf)
                    mi = jnp.where(maskc, avk, BIGF)
            else:
                pred = v > m
                if not full:
                    pred = jnp.logical_and(pred, maskc)
                m = jnp.where(pred, v, m)
                mi = jnp.where(pred, avk, mi)
        g = m
        for sh in (8, 4, 2, 1):
            g = jnp.maximum(g, _perm(g, lane ^ sh))
        cand = jnp.where(m == g, mi, BIGF)
        for sh in (8, 4, 2, 1):
            cand = jnp.minimum(cand, _perm(cand, lane ^ sh))
        si = jnp.where(lane == (s % L), cand - float(lo), si)
        if s % L == L - 1 or s == NSEG - 1:
            t = s // L
            sii = si.astype(jnp.int32)
            c_idx = sii >> 4
            w_idx = sii & (L - 1)
            r0 = _perm(tab0[0], w_idx)
            r1 = _perm(tab1[0], w_idx)
            for c in range(1, NTAB):
                hit = c_idx == c
                r0 = jnp.where(hit, _perm(tab0[c], w_idx), r0)
                r1 = jnp.where(hit, _perm(tab1[c], w_idx), r1)
            i0 = r0.astype(jnp.int32)
            i1 = r1.astype(jnp.int32)
            olo = jnp.where(even, _perm(i0, half), _perm(i1, half))
            outbuf[pl.ds(2 * L * t, L)] = olo
            if 2 * L * t + 2 * L <= OPAD:
                ohi = jnp.where(even, _perm(i0, half + 8), _perm(i1, half + 8))
                outbuf[pl.ds(2 * L * t + L, L)] = ohi


def _sc_body(x_hbm, w0_hbm, w1_hbm, msg_hbm,
             bufA, bufB, w0buf, w1buf, outA, outB,
             semA, semB, osemA, osemB):
    wid = lax.axis_index("s") * 2 + lax.axis_index("c")
    base = wid * ROWS_PER
    last = base + ROWS_PER - 1
    pltpu.sync_copy(w0_hbm, w0buf.at[pl.ds(0, N_VAL)])
    pltpu.sync_copy(w1_hbm, w1buf.at[pl.ds(0, N_VAL)])
    tab0 = [w0buf[pl.ds(L * c, L)] for c in range(NTAB)]
    tab1 = [w1buf[pl.ds(L * c, L)] for c in range(NTAB)]

    dstA = bufA.at[pl.ds(0, HALF)]
    dstB = bufB.at[pl.ds(0, HALF)]

    def srcA(r):
        return x_hbm.at[r, pl.ds(0, HALF)]

    def srcB(r):
        return x_hbm.at[r, pl.ds(HALF, HALF)]

    pltpu.async_copy(srcA(base), dstA, semA)

    def row_body(i, carry):
        r = base + i
        pltpu.async_copy(srcB(r), dstB, semB)

        pltpu.make_async_copy(srcA(r), dstA, semA).wait()
        pl.when(i > 0)(
            lambda: pltpu.make_async_copy(
                outA.at[pl.ds(0, HCPY)], msg_hbm.at[r, pl.ds(0, HCPY)],
                osemA).wait())
        _half_compute(bufA, tab0, tab1, outA)
        pltpu.async_copy(
            outA.at[pl.ds(0, HCPY)], msg_hbm.at[r, pl.ds(0, HCPY)], osemA)

        pltpu.async_copy(srcA(jnp.minimum(r + 1, last)), dstA, semA)

        pltpu.make_async_copy(srcB(r), dstB, semB).wait()
        pl.when(i > 0)(
            lambda: pltpu.make_async_copy(
                outB.at[pl.ds(0, HCPY)], msg_hbm.at[r, pl.ds(OPAD, HCPY)],
                osemB).wait())
        _half_compute(bufB, tab0, tab1, outB)
        pltpu.async_copy(
            outB.at[pl.ds(0, HCPY)], msg_hbm.at[r, pl.ds(OPAD, HCPY)], osemB)
        return carry

    lax.fori_loop(0, ROWS_PER, row_body, 0)
    pltpu.make_async_copy(srcA(last), dstA, semA).wait()
    pltpu.make_async_copy(
        outA.at[pl.ds(0, HCPY)], msg_hbm.at[last, pl.ds(0, HCPY)], osemA).wait()
    pltpu.make_async_copy(
        outB.at[pl.ds(0, HCPY)], msg_hbm.at[last, pl.ds(OPAD, HCPY)], osemB).wait()


_sc_kernel = functools.partial(
    pl.kernel,
    mesh=plsc.VectorSubcoreMesh(core_axis_name="c", subcore_axis_name="s"),
    out_type=jax.ShapeDtypeStruct((SC_ROWS, OUTP), jnp.int32),
    compiler_params=pltpu.CompilerParams(use_tc_tiling_on_sc=False),
    scratch_types=[
        pltpu.VMEM((HPAD,), jnp.float32),
        pltpu.VMEM((HPAD,), jnp.float32),
        pltpu.VMEM((NTAB * L,), jnp.float32),
        pltpu.VMEM((NTAB * L,), jnp.float32),
        pltpu.VMEM((OPAD,), jnp.int32),
        pltpu.VMEM((OPAD,), jnp.int32),
        pltpu.SemaphoreType.DMA,
        pltpu.SemaphoreType.DMA,
        pltpu.SemaphoreType.DMA,
        pltpu.SemaphoreType.DMA,
    ],
)(_sc_body)


def _tc_body(x_ref, w_ref, msg_ref):
    w = w_ref[...].astype(jnp.float32)
    for a in range(N_ATTR):
        seg = x_ref[:, a * N_VAL:(a + 1) * N_VAL]
        mx = jnp.max(seg, axis=1, keepdims=True)
        iota = lax.broadcasted_iota(jnp.int32, seg.shape, 1)
        cand = jnp.where(seg == mx, iota, N_VAL)
        idxv = jnp.min(cand, axis=1, keepdims=True)
        oh = (iota == idxv).astype(jnp.float32)
        pair = jax.lax.dot(oh, w)
        msg_ref[:, 2 * a:2 * a + 2] = pair.astype(jnp.int32)


def _tc_kernel(x, mapping_weight):
    grid = (TC_ROWS // B_BLK,)
    off = SC_ROWS // B_BLK
    return pl.pallas_call(
        _tc_body,
        grid=grid,
        in_specs=[
            pl.BlockSpec((B_BLK, ROW), lambda i: (off + i, 0)),
            pl.BlockSpec((N_VAL, 2), lambda i: (0, 0)),
        ],
        out_specs=pl.BlockSpec((B_BLK, OUT), lambda i: (i, 0)),
        out_shape=jax.ShapeDtypeStruct((TC_ROWS, OUT), jnp.int32),
    )(x, mapping_weight)


@jax.jit
def kernel(x, mapping_weight):
    w0 = mapping_weight[:, 0]
    w1 = mapping_weight[:, 1]
    msgp = _sc_kernel(x, w0, w1)
    msg_sc = jnp.concatenate(
        [msgp[:, :HOUT], msgp[:, OPAD:OPAD + HOUT]], axis=1)
    msg_tc = _tc_kernel(x, mapping_weight)
    msg = jnp.concatenate([msg_sc, msg_tc], axis=0)
    zeros = jnp.zeros((BATCH, OUT), dtype=jnp.float32)
    return (msg, zeros, zeros)


# speedup vs baseline: 1.5062x; 1.5062x over previous
import functools

import numpy as np
import jax
import jax.numpy as jnp
from jax import lax
from jax.experimental import pallas as pl
from jax.experimental.pallas import tpu as pltpu
from jax.experimental.pallas import tpu_sc as plsc

N_ATTR = 100
N_VAL = 100
ROW = N_ATTR * N_VAL
HALF = ROW // 2
NSEG = N_ATTR // 2
HOUT = NSEG * 2
HCPY = 104
BATCH = 4096
OUT = 2 * N_ATTR
OUTP = 224
L = 16
NTAB = (N_VAL + L - 1) // L
NWORKER = 32
ROWS_PER = BATCH // NWORKER
HPAD = 5008
OPAD = 112
BIGF = np.float32(2.0 ** 30)


def _perm(v, idx):
    return v.at[idx].get(mode="promise_in_bounds")


def _half_compute(buf, tab0, tab1, outbuf):
    lane = lax.iota(jnp.int32, L)
    lanef = lane.astype(jnp.float32)
    half = lane >> 1
    even = (lane & 1) == 0
    si = jnp.zeros((L,), jnp.float32)
    chunk_cache = {}

    def chunk(k):
        if k not in chunk_cache:
            chunk_cache[k] = (buf[pl.ds(L * k, L)], lanef + float(L * k))
        return chunk_cache[k]

    for s in range(NSEG):
        lo = N_VAL * s
        hi = lo + N_VAL
        k0 = lo // L
        k1 = (hi - 1) // L
        m = None
        for k in range(k0, k1 + 1):
            start = L * k
            v, avk = chunk(k)
            full = start >= lo and start + L <= hi
            maskc = None
            if not full:
                maskc = avk >= lo if start < lo else avk < hi
            if m is None:
                if full:
                    m, mi = v, avk
                else:
                    m = jnp.where(maskc, v, -jnp.inf)
                    mi = jnp.where(maskc, avk, BIGF)
            else:
                pred = v > m
                if not full:
                    pred = jnp.logical_and(pred, maskc)
                    m = jnp.where(pred, v, m)
                else:
                    m = jnp.maximum(m, v)
                mi = jnp.where(pred, avk, mi)
        g = m
        for sh in (8, 4, 2, 1):
            g = jnp.maximum(g, _perm(g, lane ^ sh))
        cand = jnp.where(m == g, mi, BIGF)
        for sh in (8, 4, 2, 1):
            cand = jnp.minimum(cand, _perm(cand, lane ^ sh))
        si = jnp.where(lane == (s % L), cand - float(lo), si)
        if s % L == L - 1 or s == NSEG - 1:
            t = s // L
            sii = si.astype(jnp.int32)
            c_idx = sii >> 4
            w_idx = sii & (L - 1)
            r0 = _perm(tab0[0], w_idx)
            r1 = _perm(tab1[0], w_idx)
            for c in range(1, NTAB):
                hit = c_idx == c
                r0 = jnp.where(hit, _perm(tab0[c], w_idx), r0)
                r1 = jnp.where(hit, _perm(tab1[c], w_idx), r1)
            i0 = r0.astype(jnp.int32)
            i1 = r1.astype(jnp.int32)
            olo = jnp.where(even, _perm(i0, half), _perm(i1, half))
            outbuf[pl.ds(2 * L * t, L)] = olo
            if 2 * L * t + 2 * L <= OPAD:
                ohi = jnp.where(even, _perm(i0, half + 8), _perm(i1, half + 8))
                outbuf[pl.ds(2 * L * t + L, L)] = ohi


def _sc_body(x_hbm, w0_hbm, w1_hbm, msg_hbm,
             bufA, bufB, w0buf, w1buf, outA, outB,
             semA, semB, osemA, osemB):
    wid = lax.axis_index("s") * 2 + lax.axis_index("c")
    base = wid * ROWS_PER
    last = base + ROWS_PER - 1
    pltpu.sync_copy(w0_hbm, w0buf.at[pl.ds(0, N_VAL)])
    pltpu.sync_copy(w1_hbm, w1buf.at[pl.ds(0, N_VAL)])
    tab0 = [w0buf[pl.ds(L * c, L)] for c in range(NTAB)]
    tab1 = [w1buf[pl.ds(L * c, L)] for c in range(NTAB)]

    dstA = bufA.at[pl.ds(0, HALF)]
    dstB = bufB.at[pl.ds(0, HALF)]

    def srcA(r):
        return x_hbm.at[r, pl.ds(0, HALF)]

    def srcB(r):
        return x_hbm.at[r, pl.ds(HALF, HALF)]

    pltpu.async_copy(srcA(base), dstA, semA)

    def row_body(i, carry):
        r = base + i
        pltpu.async_copy(srcB(r), dstB, semB)

        pltpu.make_async_copy(srcA(r), dstA, semA).wait()
        pl.when(i > 0)(
            lambda: pltpu.make_async_copy(
                outA.at[pl.ds(0, HCPY)], msg_hbm.at[r, pl.ds(0, HCPY)],
                osemA).wait())
        _half_compute(bufA, tab0, tab1, outA)
        pltpu.async_copy(
            outA.at[pl.ds(0, HCPY)], msg_hbm.at[r, pl.ds(0, HCPY)], osemA)

        pltpu.async_copy(srcA(jnp.minimum(r + 1, last)), dstA, semA)

        pltpu.make_async_copy(srcB(r), dstB, semB).wait()
        pl.when(i > 0)(
            lambda: pltpu.make_async_copy(
                outB.at[pl.ds(0, HCPY)], msg_hbm.at[r, pl.ds(OPAD, HCPY)],
                osemB).wait())
        _half_compute(bufB, tab0, tab1, outB)
        pltpu.async_copy(
            outB.at[pl.ds(0, HCPY)], msg_hbm.at[r, pl.ds(OPAD, HCPY)], osemB)
        return carry

    lax.fori_loop(0, ROWS_PER, row_body, 0)
    pltpu.make_async_copy(srcA(last), dstA, semA).wait()
    pltpu.make_async_copy(
        outA.at[pl.ds(0, HCPY)], msg_hbm.at[last, pl.ds(0, HCPY)], osemA).wait()
    pltpu.make_async_copy(
        outB.at[pl.ds(0, HCPY)], msg_hbm.at[last, pl.ds(OPAD, HCPY)], osemB).wait()


_sc_kernel = functools.partial(
    pl.kernel,
    mesh=plsc.VectorSubcoreMesh(core_axis_name="c", subcore_axis_name="s"),
    out_type=jax.ShapeDtypeStruct((BATCH, OUTP), jnp.int32),
    compiler_params=pltpu.CompilerParams(use_tc_tiling_on_sc=False),
    scratch_types=[
        pltpu.VMEM((HPAD,), jnp.float32),
        pltpu.VMEM((HPAD,), jnp.float32),
        pltpu.VMEM((NTAB * L,), jnp.float32),
        pltpu.VMEM((NTAB * L,), jnp.float32),
        pltpu.VMEM((OPAD,), jnp.int32),
        pltpu.VMEM((OPAD,), jnp.int32),
        pltpu.SemaphoreType.DMA,
        pltpu.SemaphoreType.DMA,
        pltpu.SemaphoreType.DMA,
        pltpu.SemaphoreType.DMA,
    ],
)(_sc_body)


@jax.jit
def kernel(x, mapping_weight):
    w0 = mapping_weight[:, 0]
    w1 = mapping_weight[:, 1]
    msgp = _sc_kernel(x, w0, w1)
    msg = jnp.concatenate(
        [msgp[:, :HOUT], msgp[:, OPAD:OPAD + HOUT]], axis=1)
    zeros = jnp.zeros((BATCH, OUT), dtype=jnp.float32)
    return (msg, zeros, zeros)


# speedup vs baseline: 1.6941x; 1.1247x over previous
import functools

import numpy as np
import jax
import jax.numpy as jnp
from jax import lax
from jax.experimental import pallas as pl
from jax.experimental.pallas import tpu as pltpu
from jax.experimental.pallas import tpu_sc as plsc

N_ATTR = 100
N_VAL = 100
ROW = N_ATTR * N_VAL
HALF = ROW // 2
NSEG = N_ATTR // 2
HOUT = NSEG * 2
HCPY = 104
BATCH = 4096
OUT = 2 * N_ATTR
OUTP = 224
L = 16
NTAB = (N_VAL + L - 1) // L
NWORKER = 32
ROWS_PER = BATCH // NWORKER
HPAD = 5008
OPAD = 112
BIGF = np.float32(2.0 ** 30)


def _perm(v, idx):
    return v.at[idx].get(mode="promise_in_bounds")


def _half_compute(buf, tab0, tab1, outbuf, zbuf):
    lane = lax.iota(jnp.int32, L)
    lanef = lane.astype(jnp.float32) + zbuf[...]
    half = lane >> 1
    even = (lane & 1) == 0
    si = jnp.zeros((L,), jnp.float32)
    chunk_cache = {}

    def chunk(k):
        if k not in chunk_cache:
            chunk_cache[k] = (buf[pl.ds(L * k, L)], lanef + float(L * k))
        return chunk_cache[k]

    for s in range(NSEG):
        lo = N_VAL * s
        hi = lo + N_VAL
        k0 = lo // L
        k1 = (hi - 1) // L
        m = None
        for k in range(k0, k1 + 1):
            start = L * k
            v, avk = chunk(k)
            full = start >= lo and start + L <= hi
            maskc = None
            if not full:
                maskc = avk >= lo if start < lo else avk < hi
            if m is None:
                if full:
                    m, mi = v, avk
                else:
                    m = jnp.where(maskc, v, -jnp.inf)
                    mi = jnp.where(maskc, avk, BIGF)
            else:
                pred = v > m
                if not full:
                    pred = jnp.logical_and(pred, maskc)
                    m = jnp.where(pred, v, m)
                else:
                    m = jnp.maximum(m, v)
                mi = jnp.where(pred, avk, mi)
        g = m
        for sh in (8, 4, 2, 1):
            g = jnp.maximum(g, _perm(g, lane ^ sh))
        cand = jnp.where(m == g, mi, BIGF)
        for sh in (8, 4, 2, 1):
            cand = jnp.minimum(cand, _perm(cand, lane ^ sh))
        si = jnp.where(lane == (s % L), cand - float(lo), si)
        if s % L == L - 1 or s == NSEG - 1:
            t = s // L
            sii = si.astype(jnp.int32)
            c_idx = sii >> 4
            w_idx = sii & (L - 1)
            r0 = _perm(tab0[0], w_idx)
            r1 = _perm(tab1[0], w_idx)
            for c in range(1, NTAB):
                hit = c_idx == c
                r0 = jnp.where(hit, _perm(tab0[c], w_idx), r0)
                r1 = jnp.where(hit, _perm(tab1[c], w_idx), r1)
            i0 = r0.astype(jnp.int32)
            i1 = r1.astype(jnp.int32)
            olo = jnp.where(even, _perm(i0, half), _perm(i1, half))
            outbuf[pl.ds(2 * L * t, L)] = olo
            if 2 * L * t + 2 * L <= OPAD:
                ohi = jnp.where(even, _perm(i0, half + 8), _perm(i1, half + 8))
                outbuf[pl.ds(2 * L * t + L, L)] = ohi


def _sc_body(x_hbm, w0_hbm, w1_hbm, msg_hbm,
             bufA, bufB, w0buf, w1buf, outA, outB, zbuf,
             semA, semB, osemA, osemB):
    wid = lax.axis_index("s") * 2 + lax.axis_index("c")
    base = wid * ROWS_PER
    last = base + ROWS_PER - 1
    pltpu.sync_copy(w0_hbm, w0buf.at[pl.ds(0, N_VAL)])
    pltpu.sync_copy(w1_hbm, w1buf.at[pl.ds(0, N_VAL)])
    tab0 = [w0buf[pl.ds(L * c, L)] for c in range(NTAB)]
    tab1 = [w1buf[pl.ds(L * c, L)] for c in range(NTAB)]

    dstA = bufA.at[pl.ds(0, HALF)]
    dstB = bufB.at[pl.ds(0, HALF)]

    def srcA(r):
        return x_hbm.at[r, pl.ds(0, HALF)]

    def srcB(r):
        return x_hbm.at[r, pl.ds(HALF, HALF)]

    zbuf[...] = jnp.zeros((L,), jnp.float32)
    pltpu.async_copy(srcA(base), dstA, semA)

    def row_body(i, carry):
        r = base + i
        pltpu.async_copy(srcB(r), dstB, semB)

        pltpu.make_async_copy(srcA(r), dstA, semA).wait()
        pl.when(i > 0)(
            lambda: pltpu.make_async_copy(
                outA.at[pl.ds(0, HCPY)], msg_hbm.at[r, pl.ds(0, HCPY)],
                osemA).wait())
        _half_compute(bufA, tab0, tab1, outA, zbuf)
        pltpu.async_copy(
            outA.at[pl.ds(0, HCPY)], msg_hbm.at[r, pl.ds(0, HCPY)], osemA)

        pltpu.async_copy(srcA(jnp.minimum(r + 1, last)), dstA, semA)

        pltpu.make_async_copy(srcB(r), dstB, semB).wait()
        pl.when(i > 0)(
            lambda: pltpu.make_async_copy(
                outB.at[pl.ds(0, HCPY)], msg_hbm.at[r, pl.ds(OPAD, HCPY)],
                osemB).wait())
        _half_compute(bufB, tab0, tab1, outB, zbuf)
        pltpu.async_copy(
            outB.at[pl.ds(0, HCPY)], msg_hbm.at[r, pl.ds(OPAD, HCPY)], osemB)
        return carry

    lax.fori_loop(0, ROWS_PER, row_body, 0)
    pltpu.make_async_copy(srcA(last), dstA, semA).wait()
    pltpu.make_async_copy(
        outA.at[pl.ds(0, HCPY)], msg_hbm.at[last, pl.ds(0, HCPY)], osemA).wait()
    pltpu.make_async_copy(
        outB.at[pl.ds(0, HCPY)], msg_hbm.at[last, pl.ds(OPAD, HCPY)], osemB).wait()


_sc_kernel = functools.partial(
    pl.kernel,
    mesh=plsc.VectorSubcoreMesh(core_axis_name="c", subcore_axis_name="s"),
    out_type=jax.ShapeDtypeStruct((BATCH, OUTP), jnp.int32),
    compiler_params=pltpu.CompilerParams(use_tc_tiling_on_sc=False),
    scratch_types=[
        pltpu.VMEM((HPAD,), jnp.float32),
        pltpu.VMEM((HPAD,), jnp.float32),
        pltpu.VMEM((NTAB * L,), jnp.float32),
        pltpu.VMEM((NTAB * L,), jnp.float32),
        pltpu.VMEM((OPAD,), jnp.int32),
        pltpu.VMEM((OPAD,), jnp.int32),
        pltpu.VMEM((L,), jnp.float32),
        pltpu.SemaphoreType.DMA,
        pltpu.SemaphoreType.DMA,
        pltpu.SemaphoreType.DMA,
        pltpu.SemaphoreType.DMA,
    ],
)(_sc_body)


@jax.jit
def kernel(x, mapping_weight):
    w0 = mapping_weight[:, 0]
    w1 = mapping_weight[:, 1]
    msgp = _sc_kernel(x, w0, w1)
    msg = jnp.concatenate(
        [msgp[:, :HOUT], msgp[:, OPAD:OPAD + HOUT]], axis=1)
    zeros = jnp.zeros((BATCH, OUT), dtype=jnp.float32)
    return (msg, zeros, zeros)


# speedup vs baseline: 1.8092x; 1.0680x over previous
import functools

import numpy as np
import jax
import jax.numpy as jnp
from jax import lax
from jax.experimental import pallas as pl
from jax.experimental.pallas import tpu as pltpu
from jax.experimental.pallas import tpu_sc as plsc

N_ATTR = 100
N_VAL = 100
ROW = N_ATTR * N_VAL
HALF = ROW // 2
NSEG = N_ATTR // 2
HOUT = NSEG * 2
HCPY = 104
BATCH = 4096
OUT = 2 * N_ATTR
OUTP = 224
L = 16
NTAB = (N_VAL + L - 1) // L
NWORKER = 32
ROWS_PER = BATCH // NWORKER
HPAD = 5008
OPAD = 112
BIGF = np.float32(2.0 ** 30)


def _perm(v, idx):
    return v.at[idx].get(mode="promise_in_bounds")


def _half_compute(buf, tab0, outbuf, zbuf):
    lane = lax.iota(jnp.int32, L)
    lanef = lane.astype(jnp.float32) + zbuf[...]
    half = lane >> 1
    even = (lane & 1) == 0
    si = jnp.zeros((L,), jnp.float32)
    chunk_cache = {}

    def chunk(k):
        if k not in chunk_cache:
            chunk_cache[k] = (buf[pl.ds(L * k, L)], lanef + float(L * k))
        return chunk_cache[k]

    for s in range(NSEG):
        lo = N_VAL * s
        hi = lo + N_VAL
        k0 = lo // L
        k1 = (hi - 1) // L
        m = None
        for k in range(k0, k1 + 1):
            start = L * k
            v, avk = chunk(k)
            full = start >= lo and start + L <= hi
            slot = jnp.float32(k - k0)
            maskc = None
            if not full:
                maskc = avk >= lo if start < lo else avk < hi
            if m is None:
                if full:
                    m, mi = v, jnp.zeros((L,), jnp.float32)
                else:
                    m = jnp.where(maskc, v, -jnp.inf)
                    mi = jnp.where(maskc, 0.0, BIGF)
            else:
                pred = v > m
                if not full:
                    pred = jnp.logical_and(pred, maskc)
                    m = jnp.where(pred, v, m)
                else:
                    m = jnp.maximum(m, v)
                mi = jnp.where(pred, slot, mi)
        g = m
        for sh in (8, 4, 2, 1):
            g = jnp.maximum(g, _perm(g, lane ^ sh))
        cand = jnp.where(m == g, mi * float(L) + lanef, BIGF)
        for sh in (8, 4, 2, 1):
            cand = jnp.minimum(cand, _perm(cand, lane ^ sh))
        si = jnp.where(lane == (s % L), cand + float(L * k0 - lo), si)
        if s % L == L - 1 or s == NSEG - 1:
            t = s // L
            sii = si.astype(jnp.int32)
            c_idx = sii >> 4
            w_idx = sii & (L - 1)
            rp = _perm(tab0[0], w_idx)
            for c in range(1, NTAB):
                hit = c_idx == c
                rp = jnp.where(hit, _perm(tab0[c], w_idx), rp)
            i0 = rp & (L - 1)
            i1 = rp >> 4
            olo = jnp.where(even, _perm(i0, half), _perm(i1, half))
            outbuf[pl.ds(2 * L * t, L)] = olo
            if 2 * L * t + 2 * L <= OPAD:
                ohi = jnp.where(even, _perm(i0, half + 8), _perm(i1, half + 8))
                outbuf[pl.ds(2 * L * t + L, L)] = ohi


def _sc_body(x_hbm, wp_hbm, msg_hbm,
             bufA, bufB, wpbuf, outA, outB, zbuf,
             semA, semB, osemA, osemB):
    wid = lax.axis_index("s") * 2 + lax.axis_index("c")
    base = wid * ROWS_PER
    last = base + ROWS_PER - 1
    pltpu.sync_copy(wp_hbm, wpbuf.at[pl.ds(0, N_VAL)])
    tab0 = [wpbuf[pl.ds(L * c, L)] for c in range(NTAB)]

    dstA = bufA.at[pl.ds(0, HALF)]
    dstB = bufB.at[pl.ds(0, HALF)]

    def srcA(r):
        return x_hbm.at[r, pl.ds(0, HALF)]

    def srcB(r):
        return x_hbm.at[r, pl.ds(HALF, HALF)]

    zbuf[...] = jnp.zeros((L,), jnp.float32)
    pltpu.async_copy(srcA(base), dstA, semA)

    def row_body(i, carry):
        r = base + i
        pltpu.async_copy(srcB(r), dstB, semB)

        pltpu.make_async_copy(srcA(r), dstA, semA).wait()
        pl.when(i > 0)(
            lambda: pltpu.make_async_copy(
                outA.at[pl.ds(0, HCPY)], msg_hbm.at[r, pl.ds(0, HCPY)],
                osemA).wait())
        _half_compute(bufA, tab0, outA, zbuf)
        pltpu.async_copy(
            outA.at[pl.ds(0, HCPY)], msg_hbm.at[r, pl.ds(0, HCPY)], osemA)

        pltpu.async_copy(srcA(jnp.minimum(r + 1, last)), dstA, semA)

        pltpu.make_async_copy(srcB(r), dstB, semB).wait()
        pl.when(i > 0)(
            lambda: pltpu.make_async_copy(
                outB.at[pl.ds(0, HCPY)], msg_hbm.at[r, pl.ds(OPAD, HCPY)],
                osemB).wait())
        _half_compute(bufB, tab0, outB, zbuf)
        pltpu.async_copy(
            outB.at[pl.ds(0, HCPY)], msg_hbm.at[r, pl.ds(OPAD, HCPY)], osemB)
        return carry

    lax.fori_loop(0, ROWS_PER, row_body, 0)
    pltpu.make_async_copy(srcA(last), dstA, semA).wait()
    pltpu.make_async_copy(
        outA.at[pl.ds(0, HCPY)], msg_hbm.at[last, pl.ds(0, HCPY)], osemA).wait()
    pltpu.make_async_copy(
        outB.at[pl.ds(0, HCPY)], msg_hbm.at[last, pl.ds(OPAD, HCPY)], osemB).wait()


_sc_kernel = functools.partial(
    pl.kernel,
    mesh=plsc.VectorSubcoreMesh(core_axis_name="c", subcore_axis_name="s"),
    out_type=jax.ShapeDtypeStruct((BATCH, OUTP), jnp.int32),
    compiler_params=pltpu.CompilerParams(use_tc_tiling_on_sc=False),
    scratch_types=[
        pltpu.VMEM((HPAD,), jnp.float32),
        pltpu.VMEM((HPAD,), jnp.float32),
        pltpu.VMEM((NTAB * L,), jnp.int32),
        pltpu.VMEM((OPAD,), jnp.int32),
        pltpu.VMEM((OPAD,), jnp.int32),
        pltpu.VMEM((L,), jnp.float32),
        pltpu.SemaphoreType.DMA,
        pltpu.SemaphoreType.DMA,
        pltpu.SemaphoreType.DMA,
        pltpu.SemaphoreType.DMA,
    ],
)(_sc_body)


@jax.jit
def kernel(x, mapping_weight):
    wm = mapping_weight.astype(jnp.int32)
    wpack = wm[:, 0] + L * wm[:, 1]
    msgp = _sc_kernel(x, wpack)
    msg = jnp.concatenate(
        [msgp[:, :HOUT], msgp[:, OPAD:OPAD + HOUT]], axis=1)
    zeros = jnp.zeros((BATCH, OUT), dtype=jnp.float32)
    return (msg, zeros, zeros)
